# Initial kernel scaffold; baseline (speedup 1.0000x reference)
import jax
import jax.numpy as jnp
from jax.experimental import pallas as pl


def kernel(input, weight_q, weight_r):
    raise NotImplementedError("write your pallas kernel here")


# speedup vs baseline: 14.8621x; 14.8621x over previous
import jax
import jax.numpy as jnp
from jax import lax
from jax.experimental import pallas as pl
from jax.experimental.pallas import tpu as pltpu
from jax.experimental.pallas import tpu_sc as plsc

NUM_COLLISIONS = 4
EMBED_DIM = 64
BATCH = 16384
HIST = 50

NC, NS, L = 2, 16, 16
NW = NC * NS
BAGS_PER_W = BATCH // NW
GB = 16
NG = BAGS_PER_W // GB
IDX_PER_G = GB * HIST
N_SUB = 10
SUB = IDX_PER_G // N_SUB
DV = EMBED_DIM // L

_mesh = plsc.VectorSubcoreMesh(core_axis_name="c", subcore_axis_name="s")


@jax.jit
def _qr_bag(inp_flat, weight_q, weight_r):
    @pl.kernel(
        out_type=jax.ShapeDtypeStruct((BATCH, EMBED_DIM), jnp.float32),
        mesh=_mesh,
        compiler_params=pltpu.CompilerParams(
            needs_layout_passes=False, use_tc_tiling_on_sc=False),
        scratch_types=[
            pltpu.VMEM((IDX_PER_G,), jnp.int32),
            pltpu.VMEM((N_SUB, SUB), jnp.int32),
            pltpu.VMEM((IDX_PER_G, EMBED_DIM), jnp.float32),
            pltpu.VMEM((NUM_COLLISIONS, EMBED_DIM), jnp.float32),
            pltpu.VMEM((GB, EMBED_DIM), jnp.float32),
            pltpu.SemaphoreType.DMA,
        ],
    )
    def kern(inp_hbm, wq_hbm, wr_hbm, out_hbm,
             raw_v, idxq_v, rows_v, wr_v, out_v, sem):
        wid = lax.axis_index("s") * NC + lax.axis_index("c")
        pltpu.sync_copy(wr_hbm, wr_v)
        lanes = lax.iota(jnp.int32, L)
        zf = jnp.zeros((L,), jnp.float32)

        wr_vec = [[wr_v[k, pl.ds(d * L, L)] for d in range(DV)]
                  for k in range(NUM_COLLISIONS)]

        def group_body(g, carry):
            base = wid * BAGS_PER_W + g * GB
            pltpu.sync_copy(inp_hbm.at[pl.ds(base * HIST, IDX_PER_G)], raw_v)

            def jbody(j, cnts):
                c0, c1, c2, c3 = cnts
                v = plsc.load_gather(raw_v, [lanes * HIST + j])
                q = lax.shift_right_logical(v, 2)
                row = j // (SUB // L)
                col = (j % (SUB // L)) * L
                idxq_v[row, pl.ds(col, L)] = q
                r = jnp.bitwise_and(v, 3)
                c0 = c0 + jnp.where(r == 0, 1.0, 0.0).astype(jnp.float32)
                c1 = c1 + jnp.where(r == 1, 1.0, 0.0).astype(jnp.float32)
                c2 = c2 + jnp.where(r == 2, 1.0, 0.0).astype(jnp.float32)
                c3 = c3 + jnp.where(r == 3, 1.0, 0.0).astype(jnp.float32)
                return (c0, c1, c2, c3)

            cnt = lax.fori_loop(0, HIST, jbody, (zf, zf, zf, zf))

            copies = [
                pltpu.async_copy(
                    wq_hbm.at[idxq_v.at[jj]],
                    rows_v.at[pl.ds(jj * SUB, SUB)],
                    sem,
                )
                for jj in range(N_SUB)
            ]
            for c in copies:
                c.wait()

            inv = jnp.float32(1.0 / (HIST * HIST))
            for l in range(GB):
                cs = [cnt[k][l] for k in range(NUM_COLLISIONS)]
                sr = [
                    (cs[0] * wr_vec[0][d] + cs[1] * wr_vec[1][d]
                     + cs[2] * wr_vec[2][d] + cs[3] * wr_vec[3][d]) * inv
                    for d in range(DV)
                ]

                def sbody(j, accs):
                    row = j * L + l
                    return tuple(
                        accs[d] + rows_v[row, pl.ds(d * L, L)]
                        for d in range(DV)
                    )

                accs = lax.fori_loop(0, HIST, sbody, (zf,) * DV)
                for d in range(DV):
                    out_v[l, pl.ds(d * L, L)] = accs[d] * sr[d]

            pltpu.sync_copy(out_v, out_hbm.at[pl.ds(base, GB)])
            return carry

        lax.fori_loop(0, NG, group_body, jnp.int32(0))

    return kern(inp_flat, weight_q, weight_r)


def kernel(input, weight_q, weight_r):
    return _qr_bag(input.reshape(-1), weight_q, weight_r)


# speedup vs baseline: 18.7293x; 1.2602x over previous
import jax
import jax.numpy as jnp
from jax import lax
from jax.experimental import pallas as pl
from jax.experimental.pallas import tpu as pltpu
from jax.experimental.pallas import tpu_sc as plsc

NUM_COLLISIONS = 4
EMBED_DIM = 64
BATCH = 16384
HIST = 50

NC, NS, L = 2, 16, 16
NW = NC * NS
BAGS_PER_W = BATCH // NW
GB = 16
NG = BAGS_PER_W // GB
IDX_PER_G = GB * HIST
N_SUB = 10
SUB = IDX_PER_G // N_SUB
JPR = SUB // L
DV = EMBED_DIM // L

_mesh = plsc.VectorSubcoreMesh(core_axis_name="c", subcore_axis_name="s")


@jax.jit
def _qr_bag(inp, weight_q, weight_r):
    inp_flat = jnp.concatenate(
        [inp.reshape(-1), jnp.zeros((IDX_PER_G,), jnp.int32)])

    @pl.kernel(
        out_type=jax.ShapeDtypeStruct((BATCH, EMBED_DIM), jnp.float32),
        mesh=_mesh,
        compiler_params=pltpu.CompilerParams(
            needs_layout_passes=False, use_tc_tiling_on_sc=False),
        scratch_types=[
            pltpu.VMEM((IDX_PER_G,), jnp.int32),
            pltpu.VMEM((2, N_SUB, SUB), jnp.int32),
            pltpu.VMEM((2, IDX_PER_G, EMBED_DIM), jnp.float32),
            pltpu.VMEM((NUM_COLLISIONS, EMBED_DIM), jnp.float32),
            pltpu.VMEM((GB, EMBED_DIM), jnp.float32),
            pltpu.SemaphoreType.DMA,
            pltpu.SemaphoreType.DMA,
        ],
    )
    def kern(inp_hbm, wq_hbm, wr_hbm, out_hbm,
             raw_v, idxq_v, rows_v, wr_v, out_v, sem0, sem1):
        sems = (sem0, sem1)
        wid = lax.axis_index("s") * NC + lax.axis_index("c")
        pltpu.sync_copy(wr_hbm, wr_v)
        lanes = lax.iota(jnp.int32, L)
        zf = jnp.zeros((L,), jnp.float32)

        wr_vec = [[wr_v[k, pl.ds(d * L, L)] for d in range(DV)]
                  for k in range(NUM_COLLISIONS)]

        def prep(g, buf):
            base = wid * BAGS_PER_W + g * GB
            pltpu.sync_copy(inp_hbm.at[pl.ds(base * HIST, IDX_PER_G)], raw_v)

            def jbody(j, cnts):
                c0, c1, c2, c3 = cnts
                v = plsc.load_gather(raw_v, [lanes * HIST + j])
                q = lax.shift_right_logical(v, 2)
                idxq_v[buf, j // JPR, pl.ds((j % JPR) * L, L)] = q
                r = jnp.bitwise_and(v, 3)
                c0 = c0 + jnp.where(r == 0, 1.0, 0.0).astype(jnp.float32)
                c1 = c1 + jnp.where(r == 1, 1.0, 0.0).astype(jnp.float32)
                c2 = c2 + jnp.where(r == 2, 1.0, 0.0).astype(jnp.float32)
                c3 = c3 + jnp.where(r == 3, 1.0, 0.0).astype(jnp.float32)
                return (c0, c1, c2, c3)

            cnt = lax.fori_loop(0, HIST, jbody, (zf,) * 4, unroll=5)
            for jj in range(N_SUB):
                pltpu.async_copy(
                    wq_hbm.at[idxq_v.at[buf, jj]],
                    rows_v.at[buf, pl.ds(jj * SUB, SUB)],
                    sems[buf],
                )
            return cnt

        def drain(buf):
            for jj in range(N_SUB):
                pltpu.make_async_copy(
                    wq_hbm.at[idxq_v.at[buf, jj]],
                    rows_v.at[buf, pl.ds(jj * SUB, SUB)],
                    sems[buf],
                ).wait()

        def compute(g, buf, cnt):
            base = wid * BAGS_PER_W + g * GB
            drain(buf)
            inv = jnp.float32(1.0 / (HIST * HIST))
            for l in range(GB):
                cs = [cnt[k][l] for k in range(NUM_COLLISIONS)]
                sr = [
                    (cs[0] * wr_vec[0][d] + cs[1] * wr_vec[1][d]
                     + cs[2] * wr_vec[2][d] + cs[3] * wr_vec[3][d]) * inv
                    for d in range(DV)
                ]

                def sbody(j, accs):
                    row = j * L + l
                    return tuple(
                        accs[d] + rows_v[buf, row, pl.ds(d * L, L)]
                        for d in range(DV)
                    )

                accs = lax.fori_loop(0, HIST, sbody, (zf,) * DV, unroll=5)
                for d in range(DV):
                    out_v[l, pl.ds(d * L, L)] = accs[d] * sr[d]

            pltpu.sync_copy(out_v, out_hbm.at[pl.ds(base, GB)])

        cnt0 = prep(jnp.int32(0), 0)

        def body(gg, cnt_cur):
            g0 = gg * 2
            cnt_n1 = prep(g0 + 1, 1)
            compute(g0, 0, cnt_cur)
            cnt_n2 = prep(g0 + 2, 0)
            compute(g0 + 1, 1, cnt_n1)
            return cnt_n2

        lax.fori_loop(0, NG // 2, body, cnt0)
        drain(0)

    return kern(inp_flat, weight_q, weight_r)


def kernel(input, weight_q, weight_r):
    return _qr_bag(input, weight_q, weight_r)
